# Initial kernel scaffold; baseline (speedup 1.0000x reference)
import jax
import jax.numpy as jnp
from jax.experimental import pallas as pl


def kernel(x, edge_index, emb, W, w_ih, w_hh, b_ih, b_hh):
    raise NotImplementedError("write your pallas kernel here")


# speedup vs baseline: 3.9295x; 3.9295x over previous
import functools

import jax
import jax.numpy as jnp
from jax import lax
from jax.experimental import pallas as pl
from jax.experimental.pallas import tpu as pltpu
from jax.experimental.pallas import tpu_sc as plsc

N_NODES = 10000
N_EDGES = 320000
H = 128

NC = 2
NS = 16
NW = NC * NS

CHUNK = 128

NP = 10112
NODE_CHUNKS = NP // CHUNK
ROWS_PER_TILE = NP // NS

EDGE_CHUNKS = 79
EDGES_PER_TILE = EDGE_CHUNKS * CHUNK
E_PAD = NW * EDGES_PER_TILE


def _mesh():
    return plsc.VectorSubcoreMesh(core_axis_name="c", subcore_axis_name="s",
                                  num_cores=NC, num_subcores=NS)


def _embed_sc_body(emb_hbm, x_hbm, out_hbm, idx_v, rows_v, sem):
    cid = lax.axis_index("c")
    sid = lax.axis_index("s")
    wid = cid * NS + sid
    lo = wid * NODE_CHUNKS // NW
    hi = (wid + 1) * NODE_CHUNKS // NW

    def chunk(j, carry):
        off = pl.multiple_of(j * CHUNK, CHUNK)
        pltpu.sync_copy(x_hbm.at[pl.ds(off, CHUNK)], idx_v)
        pltpu.async_copy(emb_hbm.at[idx_v], rows_v, sem).wait()
        pltpu.sync_copy(rows_v, out_hbm.at[pl.ds(off, CHUNK)])
        return carry

    lax.fori_loop(lo, hi, chunk, 0)


@functools.cache
def _embed_sc():
    return pl.kernel(
        _embed_sc_body,
        out_type=jax.ShapeDtypeStruct((NP, H), jnp.float32),
        mesh=_mesh(),
        scratch_types=[
            pltpu.VMEM((CHUNK,), jnp.int32),
            pltpu.VMEM((CHUNK, H), jnp.float32),
            pltpu.SemaphoreType.DMA,
        ],
    )


def _scatter_sc_body(m_hbm, src_hbm, dst_hbm, zero_hbm, parts_hbm,
                     src_v, dst_v, rows_v, acc, sem):
    cid = lax.axis_index("c")
    sid = lax.axis_index("s")
    wid = cid * NS + sid

    pltpu.sync_copy(zero_hbm.at[pl.ds(sid * ROWS_PER_TILE, ROWS_PER_TILE)],
                    acc.at[pl.ds(sid * ROWS_PER_TILE, ROWS_PER_TILE)])
    plsc.subcore_barrier()

    def chunk(j, carry):
        off = pl.multiple_of(wid * EDGES_PER_TILE + j * CHUNK, CHUNK)
        pltpu.sync_copy(src_hbm.at[pl.ds(off, CHUNK)], src_v)
        pltpu.sync_copy(dst_hbm.at[pl.ds(off, CHUNK)], dst_v)
        pltpu.async_copy(m_hbm.at[src_v], rows_v, sem).wait()
        pltpu.sync_copy(rows_v, acc.at[dst_v], add=True)
        return carry

    lax.fori_loop(0, EDGE_CHUNKS, chunk, 0)
    plsc.subcore_barrier()

    pltpu.sync_copy(acc.at[pl.ds(sid * ROWS_PER_TILE, ROWS_PER_TILE)],
                    parts_hbm.at[cid, pl.ds(sid * ROWS_PER_TILE, ROWS_PER_TILE)])


@functools.cache
def _scatter_sc():
    return pl.kernel(
        _scatter_sc_body,
        out_type=jax.ShapeDtypeStruct((NC, NP, H), jnp.float32),
        mesh=_mesh(),
        scratch_types=[
            pltpu.VMEM((CHUNK,), jnp.int32),
            pltpu.VMEM((CHUNK,), jnp.int32),
            pltpu.VMEM((CHUNK, H), jnp.float32),
            pltpu.VMEM_SHARED((NP, H), jnp.float32),
            pltpu.SemaphoreType.DMA,
        ],
    )


ROWS_BLK = ROWS_PER_TILE


def _mm_body(h_ref, w_ref, o_ref):
    o_ref[...] = jnp.dot(h_ref[...], w_ref[...],
                         preferred_element_type=jnp.float32)


_mm = pl.pallas_call(
    _mm_body,
    grid=(NP // ROWS_BLK,),
    in_specs=[
        pl.BlockSpec((ROWS_BLK, H), lambda i: (i, 0)),
        pl.BlockSpec((H, H), lambda i: (0, 0)),
    ],
    out_specs=pl.BlockSpec((ROWS_BLK, H), lambda i: (i, 0)),
    out_shape=jax.ShapeDtypeStruct((NP, H), jnp.float32),
)


def _gru_body(parts_ref, h_ref, wih_t_ref, whh_t_ref, bih_ref, bhh_ref,
              o_ref, *, final):
    agg = parts_ref[0] + parts_ref[1]
    h = h_ref[...]
    gi = jnp.dot(agg, wih_t_ref[...],
                 preferred_element_type=jnp.float32) + bih_ref[...]
    gh = jnp.dot(h, whh_t_ref[...],
                 preferred_element_type=jnp.float32) + bhh_ref[...]
    r = jax.nn.sigmoid(gi[:, 0:H] + gh[:, 0:H])
    z = jax.nn.sigmoid(gi[:, H:2 * H] + gh[:, H:2 * H])
    n = jnp.tanh(gi[:, 2 * H:] + r * gh[:, 2 * H:])
    hn = (1.0 - z) * n + z * h
    if final:
        hn = jnp.maximum(hn, 0.0)
    o_ref[...] = hn


def _make_gru(final):
    return pl.pallas_call(
        functools.partial(_gru_body, final=final),
        grid=(NP // ROWS_BLK,),
        in_specs=[
            pl.BlockSpec((NC, ROWS_BLK, H), lambda i: (0, i, 0)),
            pl.BlockSpec((ROWS_BLK, H), lambda i: (i, 0)),
            pl.BlockSpec((H, 3 * H), lambda i: (0, 0)),
            pl.BlockSpec((H, 3 * H), lambda i: (0, 0)),
            pl.BlockSpec((1, 3 * H), lambda i: (0, 0)),
            pl.BlockSpec((1, 3 * H), lambda i: (0, 0)),
        ],
        out_specs=pl.BlockSpec((ROWS_BLK, H), lambda i: (i, 0)),
        out_shape=jax.ShapeDtypeStruct((NP, H), jnp.float32),
    )


_gru_mid = _make_gru(False)
_gru_final = _make_gru(True)


def kernel(x, edge_index, emb, W, w_ih, w_hh, b_ih, b_hh):
    x_pad = jnp.pad(x.astype(jnp.int32), (0, NP - N_NODES))
    src = jnp.pad(edge_index[0].astype(jnp.int32), (0, E_PAD - N_EDGES))
    dst = jnp.pad(edge_index[1].astype(jnp.int32), (0, E_PAD - N_EDGES),
                  constant_values=N_NODES)
    zeros = jnp.zeros((NP, H), jnp.float32)
    wih_t = w_ih.T
    whh_t = w_hh.T
    bih = b_ih.reshape(1, 3 * H)
    bhh = b_hh.reshape(1, 3 * H)

    h = _embed_sc()(emb, x_pad)
    for i in range(W.shape[0]):
        m = _mm(h, W[i])
        parts = _scatter_sc()(m, src, dst, zeros)
        gru = _gru_final if i == W.shape[0] - 1 else _gru_mid
        h = gru(parts, h, wih_t, whh_t, bih, bhh)
    return h[:N_NODES]
